# Initial kernel scaffold; baseline (speedup 1.0000x reference)
import jax
import jax.numpy as jnp
from jax.experimental import pallas as pl


def kernel(x, edge_index, W, b):
    raise NotImplementedError("write your pallas kernel here")


# speedup vs baseline: 23.2775x; 23.2775x over previous
import functools

import jax
import jax.numpy as jnp
from jax import lax
from jax.experimental import pallas as pl
from jax.experimental.pallas import tpu as pltpu
from jax.experimental.pallas import tpu_sc as plsc

N = 10000
D = 128
C = 16
NC, NS, L = 2, 16, 16
NW = NC * NS
NPAD = 10240
ROWS_T = NPAD // NS
DUMMY = N
CHUNK = 128


def _fill_const(ref, n, val):
    def body(i, _):
        ref[pl.ds(i * L, L)] = jnp.full((L,), val, jnp.float32)
        return 0
    lax.fori_loop(0, n // L, body, 0)


def _rsqrt16(d):
    i = lax.bitcast_convert_type(d, jnp.int32)
    i = jnp.int32(0x5F3759DF) - lax.shift_right_logical(i, jnp.full((L,), 1, jnp.int32))
    y = lax.bitcast_convert_type(i, jnp.float32)
    for _ in range(3):
        y = y * (1.5 - 0.5 * d * y * y)
    return y


def _splat(vec, j):
    return jnp.broadcast_to(vec[j], (L,))


def _tc_matmul(xp, wt):
    def body(x_ref, w_ref, o_ref):
        o_ref[...] = jnp.dot(x_ref[...], w_ref[...],
                             preferred_element_type=jnp.float32)
    return pl.pallas_call(
        body,
        grid=(NPAD // 512,),
        in_specs=[pl.BlockSpec((512, D), lambda i: (i, 0)),
                  pl.BlockSpec((D, C), lambda i: (0, 0))],
        out_specs=pl.BlockSpec((512, C), lambda i: (i, 0)),
        out_shape=jax.ShapeDtypeStruct((NPAD, C), jnp.float32),
    )(xp, wt)


def _sc_deg(dst_r):
    nchunks = dst_r.shape[1]
    mesh = plsc.VectorSubcoreMesh(core_axis_name="c", subcore_axis_name="s")

    @functools.partial(
        pl.kernel,
        out_type=jax.ShapeDtypeStruct((NC, NPAD), jnp.float32),
        mesh=mesh,
        compiler_params=pltpu.CompilerParams(use_tc_tiling_on_sc=False),
        scratch_types=[
            pltpu.VMEM((CHUNK,), jnp.int32),
            pltpu.VMEM((CHUNK,), jnp.float32),
            pltpu.VMEM((ROWS_T,), jnp.float32),
            pltpu.VMEM_SHARED((NPAD,), jnp.float32),
        ],
    )
    def k(dst_hbm, deg_out, idx_v, ones_v, buf_v, acc):
        c = lax.axis_index("c")
        s = lax.axis_index("s")
        w = c * NS + s
        _fill_const(ones_v, CHUNK, 1.0)
        _fill_const(buf_v, ROWS_T, 0.5)
        pltpu.sync_copy(buf_v, acc.at[pl.ds(s * ROWS_T, ROWS_T)])
        plsc.subcore_barrier()

        def body(j, _):
            pltpu.sync_copy(dst_hbm.at[w, j], idx_v)
            pltpu.sync_copy(ones_v, acc.at[idx_v], add=True)
            return 0
        lax.fori_loop(0, nchunks, body, 0)
        plsc.subcore_barrier()
        pltpu.sync_copy(acc.at[pl.ds(s * ROWS_T, ROWS_T)],
                        deg_out.at[c, pl.ds(s * ROWS_T, ROWS_T)])

    return k(dst_r)


def _sc_hop(first_hop, uin, deg, src_r, dst_r):
    nchunks = src_r.shape[1]
    mesh = plsc.VectorSubcoreMesh(core_axis_name="c", subcore_axis_name="s")
    scratch = [
        pltpu.VMEM((ROWS_T,), jnp.float32),
        pltpu.VMEM((ROWS_T,), jnp.float32),
        pltpu.VMEM((ROWS_T, C), jnp.float32),
        pltpu.VMEM((ROWS_T, C), jnp.float32),
        pltpu.VMEM((ROWS_T, C), jnp.float32),
        pltpu.VMEM((ROWS_T, C), jnp.float32),
        pltpu.VMEM((CHUNK,), jnp.int32),
        pltpu.VMEM((CHUNK,), jnp.int32),
        pltpu.VMEM((CHUNK, C), jnp.float32),
        pltpu.VMEM_SHARED((NPAD, C), jnp.float32),
        pltpu.SemaphoreType.DMA,
    ]
    out_type = (jax.ShapeDtypeStruct((NPAD, C), jnp.float32),
                jax.ShapeDtypeStruct((NC, NPAD, C), jnp.float32))

    @functools.partial(
        pl.kernel, out_type=out_type, mesh=mesh, scratch_types=scratch,
        compiler_params=pltpu.CompilerParams(use_tc_tiling_on_sc=False))
    def k(uin_hbm, deg_hbm, src_hbm, dst_hbm, u_out, v_out,
          a_v, b_v, za_v, zb_v, u_v, h_v, idx_s, idx_d, rows, acc, sem):
        c = lax.axis_index("c")
        s = lax.axis_index("s")
        w = c * NS + s
        rs = s * ROWS_T
        pltpu.sync_copy(deg_hbm.at[0, pl.ds(rs, ROWS_T)], a_v)
        pltpu.sync_copy(deg_hbm.at[1, pl.ds(rs, ROWS_T)], b_v)
        if first_hop:
            pltpu.sync_copy(uin_hbm.at[pl.ds(rs, ROWS_T)], za_v)
        else:
            pltpu.sync_copy(uin_hbm.at[0, pl.ds(rs, ROWS_T)], za_v)
            pltpu.sync_copy(uin_hbm.at[1, pl.ds(rs, ROWS_T)], zb_v)

        def scale_rows(kk, _):
            d = a_v[pl.ds(kk * L, L)] + b_v[pl.ds(kk * L, L)]
            f = _rsqrt16(d) if first_hop else (1.0 / d)
            for j in range(L):
                n = kk * L + j
                if first_hop:
                    row = za_v[n] * _splat(f, j)
                else:
                    row = (za_v[n] + zb_v[n]) * _splat(f, j)
                u_v[n] = row
                h_v[n] = row * 0.5
            return 0
        lax.fori_loop(0, ROWS_T // L, scale_rows, 0)
        pltpu.sync_copy(u_v, u_out.at[pl.ds(rs, ROWS_T)])
        pltpu.sync_copy(h_v, acc.at[pl.ds(rs, ROWS_T)])
        plsc.subcore_barrier()

        def body(j, _):
            pltpu.sync_copy(src_hbm.at[w, j], idx_s)
            pltpu.async_copy(u_out.at[idx_s], rows, sem).wait()
            pltpu.sync_copy(dst_hbm.at[w, j], idx_d)
            pltpu.sync_copy(rows, acc.at[idx_d], add=True)
            return 0
        lax.fori_loop(0, nchunks, body, 0)
        plsc.subcore_barrier()
        pltpu.sync_copy(acc.at[pl.ds(rs, ROWS_T)],
                        v_out.at[c, pl.ds(rs, ROWS_T)])

    return k(uin, deg, src_r, dst_r)


def _tc_head(v2a, v2b, dega, degb, b2):
    def body(va, vb, da, db, b_ref, o_ref):
        d = da[...] + db[...]
        wrow = lax.rsqrt(d) * (va[...] + vb[...])
        y = wrow + b_ref[...]
        m = jnp.max(y, axis=1, keepdims=True)
        e = jnp.exp(y - m)
        o_ref[...] = (y - m) - jnp.log(jnp.sum(e, axis=1, keepdims=True))
    return pl.pallas_call(
        body,
        grid=(NPAD // 512,),
        in_specs=[pl.BlockSpec((512, C), lambda i: (i, 0)),
                  pl.BlockSpec((512, C), lambda i: (i, 0)),
                  pl.BlockSpec((512, 1), lambda i: (i, 0)),
                  pl.BlockSpec((512, 1), lambda i: (i, 0)),
                  pl.BlockSpec((1, C), lambda i: (0, 0))],
        out_specs=pl.BlockSpec((512, C), lambda i: (i, 0)),
        out_shape=jax.ShapeDtypeStruct((NPAD, C), jnp.float32),
    )(v2a, v2b, dega, degb, b2)


def kernel(x, edge_index, W, b):
    E = edge_index.shape[1]
    epw = ((E + NW - 1) // NW + CHUNK - 1) // CHUNK * CHUNK
    epad = NW * epw - E
    src = edge_index[0].astype(jnp.int32)
    dst = edge_index[1].astype(jnp.int32)
    pad = jnp.full((epad,), DUMMY, jnp.int32)
    src_r = jnp.concatenate([src, pad]).reshape(NW, epw // CHUNK, CHUNK)
    dst_r = jnp.concatenate([dst, pad]).reshape(NW, epw // CHUNK, CHUNK)
    xp = jnp.pad(x, ((0, NPAD - N), (0, 0)))

    z = _tc_matmul(xp, W.T)
    deg = _sc_deg(dst_r)
    _, v1 = _sc_hop(True, z, deg, src_r, dst_r)
    _, v2 = _sc_hop(False, v1, deg, src_r, dst_r)
    out = _tc_head(v2[0], v2[1], deg[0][:, None], deg[1][:, None], b[None, :])
    return out[:N]
